# Initial kernel scaffold; baseline (speedup 1.0000x reference)
import jax
import jax.numpy as jnp
from jax.experimental import pallas as pl


def kernel(hidden_states, router_w, gate_up_proj, down_proj):
    raise NotImplementedError("write your pallas kernel here")


# speedup vs baseline: 7.9775x; 7.9775x over previous
import functools

import jax
import jax.numpy as jnp
from jax.experimental import pallas as pl
from jax.experimental.pallas import tpu as pltpu

E = 8
K = 2
ALPHA = 1.702
LIMIT = 7.0

BM = 256
NB = 24
PCAP = NB * BM
SENT = 1 << 20

_INTERPRET = False


def _router_body(x_ref, rw_ref, ti_ref, tw_ref):
    x = x_ref[...]
    logits = jnp.dot(x, rw_ref[...], preferred_element_type=jnp.float32)
    t, e = logits.shape
    eio = jax.lax.broadcasted_iota(jnp.int32, (t, e), 1)
    m1 = jnp.max(logits, axis=1, keepdims=True)
    i1 = jnp.min(jnp.where(logits == m1, eio, e), axis=1, keepdims=True)
    masked = jnp.where(eio == i1, -jnp.inf, logits)
    m2 = jnp.max(masked, axis=1, keepdims=True)
    i2 = jnp.min(jnp.where(masked == m2, eio, e), axis=1, keepdims=True)
    w1 = 1.0 / (1.0 + jnp.exp(m2 - m1))
    ti_ref[...] = jnp.concatenate([i1, i2], axis=1)
    tw_ref[...] = jnp.concatenate([w1, 1.0 - w1], axis=1)


def _router(x, router_w):
    t = x.shape[0]
    return pl.pallas_call(
        _router_body,
        out_shape=(
            jax.ShapeDtypeStruct((t, K), jnp.int32),
            jax.ShapeDtypeStruct((t, K), jnp.float32),
        ),
        interpret=_INTERPRET,
    )(x, router_w)


def _dispatch(topk_i, topk_w, x):
    t = x.shape[0]
    ef = topk_i.reshape(-1)
    wf = topk_w.reshape(-1)
    tk = ef.shape[0]
    counts = jnp.bincount(ef, length=E)
    padded = ((counts + BM - 1) // BM) * BM
    s_pad = jnp.concatenate([jnp.zeros((1,), jnp.int32),
                             jnp.cumsum(padded)[:-1].astype(jnp.int32)])
    s_cmp = jnp.concatenate([jnp.zeros((1,), jnp.int32),
                             jnp.cumsum(counts)[:-1].astype(jnp.int32)])
    order = jnp.argsort(ef, stable=True)
    es = ef[order]
    rank = jnp.arange(tk, dtype=jnp.int32) - s_cmp[es]
    pos = s_pad[es] + rank
    row_ids = jnp.full((PCAP,), SENT, jnp.int32).at[pos].set(
        (order // K).astype(jnp.int32))
    w_sorted = jnp.zeros((PCAP,), jnp.float32).at[pos].set(wf[order])
    ends = (s_pad + padded.astype(jnp.int32))
    bstart = jnp.arange(32, dtype=jnp.int32) * BM
    block_expert = jnp.minimum(
        jnp.sum(bstart[:, None] >= ends[None, :], axis=1), E - 1
    ).astype(jnp.int32)
    total = jnp.sum(padded).astype(jnp.int32)
    block_active = (bstart < total).astype(jnp.int32)
    x_sorted = x[jnp.minimum(row_ids, t - 1)]
    return x_sorted, row_ids, w_sorted, block_expert, block_active


def _mlp1_body(be_ref, ba_ref, x_ref, w_ref, act_ref):
    m = pl.program_id(0)

    @pl.when(ba_ref[m] == 1)
    def _():
        x = x_ref[...]
        w = w_ref[0]
        gu = jnp.dot(x, w, preferred_element_type=jnp.float32)
        gate = jnp.minimum(gu, LIMIT)
        up = jnp.clip(jnp.roll(gu, -1, axis=1), -LIMIT, LIMIT)
        glu = gate / (1.0 + jnp.exp(-ALPHA * gate))
        act_i = (up + 1.0) * glu
        n2 = gu.shape[1]
        sel = (jax.lax.broadcasted_iota(jnp.int32, (n2, n2 // 2), 0)
               == 2 * jax.lax.broadcasted_iota(jnp.int32, (n2, n2 // 2), 1)
               ).astype(jnp.float32)
        act_ref[...] = jnp.dot(act_i, sel, preferred_element_type=jnp.float32)

    @pl.when(ba_ref[m] == 0)
    def _():
        act_ref[...] = jnp.zeros_like(act_ref)


def _mlp1(x_sorted, gate_up, block_expert, block_active):
    h = x_sorted.shape[1]
    f2 = gate_up.shape[2]
    f = f2 // 2
    fb = 512
    nf = f // fb
    grid = (NB, nf)
    return pl.pallas_call(
        _mlp1_body,
        grid_spec=pltpu.PrefetchScalarGridSpec(
            num_scalar_prefetch=2,
            grid=grid,
            in_specs=[
                pl.BlockSpec((BM, h), lambda m, fi, be, ba: (m, 0)),
                pl.BlockSpec((1, h, 2 * fb),
                             lambda m, fi, be, ba: (be[m], 0, fi)),
            ],
            out_specs=pl.BlockSpec((BM, fb), lambda m, fi, be, ba: (m, fi)),
        ),
        out_shape=jax.ShapeDtypeStruct((PCAP, f), jnp.float32),
        interpret=_INTERPRET,
    )(block_expert, block_active, x_sorted, gate_up)


def _mlp2_body(be_ref, ba_ref, a_ref, w_ref, y_ref):
    m = pl.program_id(0)

    @pl.when(ba_ref[m] == 1)
    def _():
        y_ref[...] = jnp.dot(a_ref[...], w_ref[0],
                             preferred_element_type=jnp.float32)

    @pl.when(ba_ref[m] == 0)
    def _():
        y_ref[...] = jnp.zeros_like(y_ref)


def _mlp2(act, down, block_expert, block_active):
    f = act.shape[1]
    h = down.shape[2]
    hb = 1024
    nh = h // hb
    grid = (NB, nh)
    return pl.pallas_call(
        _mlp2_body,
        grid_spec=pltpu.PrefetchScalarGridSpec(
            num_scalar_prefetch=2,
            grid=grid,
            in_specs=[
                pl.BlockSpec((BM, f), lambda m, hi, be, ba: (m, 0)),
                pl.BlockSpec((1, f, hb),
                             lambda m, hi, be, ba: (be[m], 0, hi)),
            ],
            out_specs=pl.BlockSpec((BM, hb), lambda m, hi, be, ba: (m, hi)),
        ),
        out_shape=jax.ShapeDtypeStruct((PCAP, h), jnp.float32),
        interpret=_INTERPRET,
    )(block_expert, block_active, act, down)


def _combine_body(ids_ref, w_ref, y_ref, out_ref):
    tb = pl.program_id(0)
    rb = pl.program_id(1)

    @pl.when(rb == 0)
    def _():
        out_ref[...] = jnp.zeros_like(out_ref)

    ids = ids_ref[0, 0, :]
    w = w_ref[0, 0, :]
    tio = jax.lax.broadcasted_iota(jnp.int32, (BM, BM), 0) + tb * BM
    q = jnp.where(ids[None, :] == tio, w[None, :], 0.0)
    out_ref[...] += jnp.dot(q, y_ref[...], preferred_element_type=jnp.float32)


def _combine(row_ids, w_sorted, y, t):
    h = y.shape[1]
    nt = t // BM
    grid = (nt, NB)
    return pl.pallas_call(
        _combine_body,
        grid=grid,
        in_specs=[
            pl.BlockSpec((1, 1, BM), lambda ti, ri: (ri, 0, 0)),
            pl.BlockSpec((1, 1, BM), lambda ti, ri: (ri, 0, 0)),
            pl.BlockSpec((BM, h), lambda ti, ri: (ri, 0)),
        ],
        out_specs=pl.BlockSpec((BM, h), lambda ti, ri: (ti, 0)),
        out_shape=jax.ShapeDtypeStruct((t, h), jnp.float32),
        interpret=_INTERPRET,
    )(row_ids.reshape(NB, 1, BM), w_sorted.reshape(NB, 1, BM), y)


def kernel(hidden_states, router_w, gate_up_proj, down_proj):
    b, s, h = hidden_states.shape
    t = b * s
    x = hidden_states.reshape(t, h)
    topk_i, topk_w = _router(x, router_w)
    x_sorted, row_ids, w_sorted, be, ba = _dispatch(topk_i, topk_w, x)
    act = _mlp1(x_sorted, gate_up_proj, be, ba)
    y = _mlp2(act, down_proj, be, ba)
    out = _combine(row_ids, w_sorted, y, t)
    return out.reshape(b, s, h)
